# baseline (speedup 1.0000x reference)
import functools

import jax
import jax.numpy as jnp
from jax import lax
from jax.experimental import pallas as pl
from jax.experimental.pallas import tpu as pltpu
from jax.experimental.pallas import tpu_sc as plsc

NN = 10000
EE = 320000
HID = 128
DE = 16
XP = 16
XF = 4 * NN + 16

NC, NS = 2, 16
NW = NC * NS
EPW = EE // NW
CHUNK = 80
NCHUNK = EPW // CHUNK
S0 = 624
S_LAST = NN - S0 * (NS - 1)
ZROWS = 64

BE = 512
BN = 1000

_f32 = jnp.float32


@functools.lru_cache(maxsize=None)
def _sc_mesh():
    return plsc.VectorSubcoreMesh(
        core_axis_name="c", subcore_axis_name="s",
        num_cores=NC, num_subcores=NS)


def _gather_body(h_hbm, row_hbm, col_hbm, xf_hbm, gr_hbm, gc_hbm, cd_hbm,
                 idxr0, idxc0, bufR0, bufC0, cdb0,
                 idxr1, idxc1, bufR1, bufC1, cdb1, xvm,
                 semi0, semg0, semw0, semi1, semg1, semw1):
    cid = lax.axis_index("c")
    sid = lax.axis_index("s")
    wid = sid * NC + cid
    base = wid * EPW

    pltpu.sync_copy(xf_hbm, xvm)
    iota = lax.iota(jnp.int32, 16)
    lt3 = iota < 3

    sets = ((idxr0, idxc0, bufR0, bufC0, cdb0, semi0, semg0, semw0),
            (idxr1, idxc1, bufR1, bufC1, cdb1, semi1, semg1, semw1))

    def fire_idx(s, k):
        idxr, idxc, bufR, bufC, cdb, semi, semg, semw = sets[s]
        e0 = base + k * CHUNK
        i1 = pltpu.async_copy(row_hbm.at[pl.ds(e0, CHUNK)], idxr, semi)
        i2 = pltpu.async_copy(col_hbm.at[pl.ds(e0, CHUNK)], idxc, semi)
        return i1, i2

    def fire_gather(s):
        idxr, idxc, bufR, bufC, cdb, semi, semg, semw = sets[s]
        pltpu.async_copy(h_hbm.at[idxr], bufR, semg)
        pltpu.async_copy(h_hbm.at[idxc], bufC, semg)

    def fire_idx_and_gather(s, k):
        i1, i2 = fire_idx(s, k)
        i1.wait(); i2.wait()
        fire_gather(s)

    def wait_gather(s):
        idxr, idxc, bufR, bufC, cdb, semi, semg, semw = sets[s]
        pltpu.make_async_copy(h_hbm.at[idxr], bufR, semg).wait()
        pltpu.make_async_copy(h_hbm.at[idxc], bufC, semg).wait()

    def compute_and_write(s, k):
        idxr, idxc, bufR, bufC, cdb, semi, semg, semw = sets[s]
        e0 = base + k * CHUNK

        def egroup(g, c):
            rv = idxr[pl.ds(g * 16, 16)]
            cv = idxc[pl.ds(g * 16, 16)]
            for j in range(16):
                xr = plsc.load_gather(xvm, [rv[j] * 4 + iota])
                xc = plsc.load_gather(xvm, [cv[j] * 4 + iota])
                cdb[g * 16 + j, :] = jnp.where(lt3, xr - xc, 0.0)
            return c
        lax.fori_loop(0, CHUNK // 16, egroup, 0)

        w1 = pltpu.async_copy(bufR, gr_hbm.at[pl.ds(e0, CHUNK)], semw)
        w2 = pltpu.async_copy(bufC, gc_hbm.at[pl.ds(e0, CHUNK)], semw)
        w3 = pltpu.async_copy(cdb, cd_hbm.at[pl.ds(e0, CHUNK)], semw)
        return w1, w2, w3

    fire_idx_and_gather(0, 0)
    fire_idx_and_gather(1, 1)

    def pair(ko, carry):
        a = 2 * ko
        for s in range(2):
            k = a + s
            wait_gather(s)
            ws = compute_and_write(s, k)

            @pl.when(k + 2 < NCHUNK)
            def _():
                i1, i2 = fire_idx(s, k + 2)
                for w in ws:
                    w.wait()
                i1.wait(); i2.wait()
                fire_gather(s)

            @pl.when(k + 2 >= NCHUNK)
            def _():
                for w in ws:
                    w.wait()
        return carry

    lax.fori_loop(0, NCHUNK // 2, pair, 0)

    wait_gather(0)
    for w in compute_and_write(0, NCHUNK - 1):
        w.wait()


@functools.lru_cache(maxsize=None)
def _build_gather():
    bufset = [
        pltpu.VMEM((CHUNK,), jnp.int32),
        pltpu.VMEM((CHUNK,), jnp.int32),
        pltpu.VMEM((CHUNK, HID), _f32),
        pltpu.VMEM((CHUNK, HID), _f32),
        pltpu.VMEM((CHUNK, XP), _f32),
    ]
    return pl.kernel(
        _gather_body,
        out_type=(jax.ShapeDtypeStruct((EE, HID), _f32),
                  jax.ShapeDtypeStruct((EE, HID), _f32),
                  jax.ShapeDtypeStruct((EE, XP), _f32)),
        mesh=_sc_mesh(),
        scratch_types=bufset + bufset + [
            pltpu.VMEM((XF,), _f32),
            pltpu.SemaphoreType.DMA,
            pltpu.SemaphoreType.DMA,
            pltpu.SemaphoreType.DMA,
            pltpu.SemaphoreType.DMA,
            pltpu.SemaphoreType.DMA,
            pltpu.SemaphoreType.DMA,
        ],
        compiler_params=pltpu.CompilerParams(needs_layout_passes=False),
    )


def _sc_gather(h, row, col, xflat):
    return _build_gather()(h, row, col, xflat)


def _scatter_m_body(m_hbm, row_hbm, aggp_hbm, idxv, mbuf, idxv1, mbuf1,
                    zb, shA, sem, sem1, sems, sems1):
    cid = lax.axis_index("c")
    sid = lax.axis_index("s")
    wid = sid * NC + cid
    base = wid * EPW
    r0 = sid * S0

    z16 = jnp.zeros((16,), _f32)

    def zrow(i, c):
        for j in range(HID // 16):
            zb[i, pl.ds(j * 16, 16)] = z16
        return c
    lax.fori_loop(0, ZROWS, zrow, 0)
    nfull = jnp.where(sid == NS - 1, S_LAST // ZROWS, S0 // ZROWS)

    def zcopy(q, c):
        pltpu.sync_copy(zb, shA.at[pl.ds(r0 + q * ZROWS, ZROWS)])
        return c
    lax.fori_loop(0, nfull, zcopy, 0)

    @pl.when(sid < NS - 1)
    def _():
        tail = S0 - (S0 // ZROWS) * ZROWS
        t0 = r0 + (S0 // ZROWS) * ZROWS
        pltpu.sync_copy(zb.at[pl.ds(0, tail)], shA.at[pl.ds(t0, tail)])

    plsc.subcore_barrier()

    sets = ((idxv, mbuf, sem, sems), (idxv1, mbuf1, sem1, sems1))

    def fire_load(s, k):
        iv, mb, sm, ss = sets[s]
        e0 = base + k * CHUNK
        pltpu.async_copy(row_hbm.at[pl.ds(e0, CHUNK)], iv, sm)
        pltpu.async_copy(m_hbm.at[pl.ds(e0, CHUNK)], mb, sm)

    def wait_load(s):
        iv, mb, sm, ss = sets[s]
        pltpu.make_async_copy(row_hbm.at[pl.ds(0, CHUNK)], iv, sm).wait()
        pltpu.make_async_copy(m_hbm.at[pl.ds(0, CHUNK)], mb, sm).wait()

    def fire_scatter(s):
        iv, mb, sm, ss = sets[s]
        pltpu.async_copy(mb, shA.at[iv], ss, add=True)

    def wait_scatter(s):
        iv, mb, sm, ss = sets[s]
        pltpu.make_async_copy(mb, shA.at[iv], ss).wait()

    fire_load(0, 0)
    fire_load(1, 1)

    def pair(ko, carry):
        a = 2 * ko
        for s in range(2):
            wait_load(s)
            fire_scatter(s)
        for s in range(2):
            k = a + s

            @pl.when(k + 2 < NCHUNK)
            def _():
                wait_scatter(s)
                fire_load(s, k + 2)

            @pl.when(k + 2 >= NCHUNK)
            def _():
                wait_scatter(s)
        return carry
    lax.fori_loop(0, NCHUNK // 2, pair, 0)

    wait_load(0)
    fire_scatter(0)
    wait_scatter(0)

    plsc.subcore_barrier()

    @pl.when(sid < NS - 1)
    def _():
        sl = pl.ds(r0, S0)
        pltpu.sync_copy(shA.at[sl], aggp_hbm.at[cid, sl])

    @pl.when(sid == NS - 1)
    def _():
        sl = pl.ds(r0, S_LAST)
        pltpu.sync_copy(shA.at[sl], aggp_hbm.at[cid, sl])


@functools.lru_cache(maxsize=None)
def _build_scatter_m():
    return pl.kernel(
        _scatter_m_body,
        out_type=jax.ShapeDtypeStruct((NC, NN, HID), _f32),
        mesh=_sc_mesh(),
        scratch_types=[
            pltpu.VMEM((CHUNK,), jnp.int32),
            pltpu.VMEM((CHUNK, HID), _f32),
            pltpu.VMEM((CHUNK,), jnp.int32),
            pltpu.VMEM((CHUNK, HID), _f32),
            pltpu.VMEM((ZROWS, HID), _f32),
            pltpu.VMEM_SHARED((NN, HID), _f32),
            pltpu.SemaphoreType.DMA,
            pltpu.SemaphoreType.DMA,
            pltpu.SemaphoreType.DMA,
            pltpu.SemaphoreType.DMA,
        ],
    )


def _scatter_t_body(t_hbm, row_hbm, tp_hbm, idxv, tbuf, idxv1, tbuf1,
                    acc, sem, sem1):
    cid = lax.axis_index("c")
    sid = lax.axis_index("s")
    wid = sid * NC + cid
    base = wid * EPW

    z16 = jnp.zeros((16,), _f32)

    def zrow(i, c):
        acc[pl.ds(i * 16, 16)] = z16
        return c
    lax.fori_loop(0, XF // 16, zrow, 0)

    iota = lax.iota(jnp.int32, 16)
    lt4 = iota < 4

    sets = ((idxv, tbuf, sem), (idxv1, tbuf1, sem1))

    def fire_load(s, k):
        iv, tb, sm = sets[s]
        e0 = base + k * CHUNK
        pltpu.async_copy(row_hbm.at[pl.ds(e0, CHUNK)], iv, sm)
        pltpu.async_copy(t_hbm.at[pl.ds(e0, CHUNK)], tb, sm)

    def wait_load(s):
        iv, tb, sm = sets[s]
        pltpu.make_async_copy(row_hbm.at[pl.ds(0, CHUNK)], iv, sm).wait()
        pltpu.make_async_copy(t_hbm.at[pl.ds(0, CHUNK)], tb, sm).wait()

    def do_scatter(s):
        iv, tb, sm = sets[s]

        def egroup(g, c):
            rv = iv[pl.ds(g * 16, 16)]
            for j in range(16):
                plsc.addupdate_scatter(
                    acc, [rv[j] * 4 + iota], tb[g * 16 + j, :], mask=lt4)
            return c
        lax.fori_loop(0, CHUNK // 16, egroup, 0)

    fire_load(0, 0)
    fire_load(1, 1)

    def pair(ko, carry):
        a = 2 * ko
        for s in range(2):
            k = a + s
            wait_load(s)
            do_scatter(s)

            @pl.when(k + 2 < NCHUNK)
            def _():
                fire_load(s, k + 2)
        return carry
    lax.fori_loop(0, NCHUNK // 2, pair, 0)

    wait_load(0)
    do_scatter(0)

    pltpu.sync_copy(acc, tp_hbm.at[wid])


@functools.lru_cache(maxsize=None)
def _build_scatter_t():
    return pl.kernel(
        _scatter_t_body,
        out_type=jax.ShapeDtypeStruct((NW, XF), _f32),
        mesh=_sc_mesh(),
        scratch_types=[
            pltpu.VMEM((CHUNK,), jnp.int32),
            pltpu.VMEM((CHUNK, XP), _f32),
            pltpu.VMEM((CHUNK,), jnp.int32),
            pltpu.VMEM((CHUNK, XP), _f32),
            pltpu.VMEM((XF,), _f32),
            pltpu.SemaphoreType.DMA,
            pltpu.SemaphoreType.DMA,
        ],
        compiler_params=pltpu.CompilerParams(needs_layout_passes=False),
    )


def _sc_scatter(m, t, row):
    aggp = _build_scatter_m()(m, row)
    tp = _build_scatter_t()(t, row)
    return aggp, tp


def _silu(v):
    return v * jax.nn.sigmoid(v)


def _edge_body(gr_ref, gc_ref, cd_ref, ea_ref, w1, b1, w2, b2, wa8, ba,
               wc1, bc1, wc28, m_out, t_out):
    gr = gr_ref[...]
    gc = gc_ref[...]
    cd = cd_ref[...]
    ea = ea_ref[...]
    radial = jnp.sum(cd * cd, axis=1, keepdims=True)
    m1 = jnp.concatenate([gr, gc, radial, ea], axis=1)
    m = _silu(jnp.dot(m1, w1[...], preferred_element_type=_f32) + b1[...])
    m = _silu(jnp.dot(m, w2[...], preferred_element_type=_f32) + b2[...])
    att = jax.nn.sigmoid(
        jnp.dot(m, wa8[...], preferred_element_type=_f32)[:, :1]
        + ba[...][:, :1])
    mo = m * att
    phi = _silu(jnp.dot(mo, wc1[...], preferred_element_type=_f32) + bc1[...])
    p2 = jnp.tanh(
        jnp.dot(phi, wc28[...], preferred_element_type=_f32)[:, :1])
    t = cd * p2
    lane = lax.broadcasted_iota(jnp.int32, t.shape, 1)
    t = jnp.where(lane == 3, 1.0, t)
    m_out[...] = mo
    t_out[...] = t


def _edge_call(gr, gc, cd, ea, w1, b1, w2, b2, wa8, ba, wc1, bc1, wc28):
    eb = lambda w: pl.BlockSpec((BE, w), lambda i: (i, 0))
    fb = lambda a, b: pl.BlockSpec((a, b), lambda i: (0, 0))
    return pl.pallas_call(
        _edge_body,
        grid=(EE // BE,),
        in_specs=[eb(HID), eb(HID), eb(XP), eb(DE),
                  fb(2 * HID + 1 + DE, HID), fb(1, HID), fb(HID, HID),
                  fb(1, HID), fb(HID, 8), fb(1, HID), fb(HID, HID),
                  fb(1, HID), fb(HID, 8)],
        out_specs=[eb(HID), eb(XP)],
        out_shape=[jax.ShapeDtypeStruct((EE, HID), _f32),
                   jax.ShapeDtypeStruct((EE, XP), _f32)],
    )(gr, gc, cd, ea, w1, b1, w2, b2, wa8, ba, wc1, bc1, wc28)


def _pre_body(h_ref, win, bin_, h_out):
    h_out[...] = jnp.dot(h_ref[...], win[...],
                         preferred_element_type=_f32) + bin_[...]


def _pre_call(h, win, bin_):
    nb = lambda: pl.BlockSpec((BN, HID), lambda i: (i, 0))
    fb = lambda a: pl.BlockSpec((a, HID), lambda i: (0, 0))
    return pl.pallas_call(
        _pre_body,
        grid=(NN // BN,),
        in_specs=[nb(), fb(HID), fb(1)],
        out_specs=nb(),
        out_shape=jax.ShapeDtypeStruct((NN, HID), _f32),
    )(h, win, bin_)


def _node_body(final, h_ref, xp_ref, aggp_ref, tp_ref,
               wn1, bn1, wn2, bn2, wo, bo, *outs):
    h = h_ref[...]
    xp = xp_ref[...]
    aggp = aggp_ref[...]
    agg = aggp[0] + aggp[1]
    s4 = jnp.sum(tp_ref[...], axis=0)
    cnt = jnp.clip(s4[:, 3:4], 1.0, None)
    upd3 = s4[:, :3] / cnt
    xn = xp + jnp.concatenate(
        [upd3, jnp.zeros((upd3.shape[0], XP - 3), _f32)], axis=1)
    z1 = jnp.concatenate([h, agg], axis=1)
    z = _silu(jnp.dot(z1, wn1[...], preferred_element_type=_f32) + bn1[...])
    z = jnp.dot(z, wn2[...], preferred_element_type=_f32) + bn2[...]
    hn = h + z
    h_out, x_out = outs
    if final:
        hn = jnp.dot(hn, wo[...], preferred_element_type=_f32) + bo[...]
    h_out[...] = hn
    x_out[...] = xn


def _node_call(final, h, xp, aggp, tp3, wn1, bn1, wn2, bn2, wo, bo):
    nb = lambda w: pl.BlockSpec((BN, w), lambda i: (i, 0))
    fb = lambda a, b: pl.BlockSpec((a, b), lambda i: (0, 0))
    return pl.pallas_call(
        functools.partial(_node_body, final),
        grid=(NN // BN,),
        in_specs=[nb(HID), nb(XP),
                  pl.BlockSpec((NC, BN, HID), lambda i: (0, i, 0)),
                  pl.BlockSpec((NW, BN, 4), lambda i: (0, i, 0)),
                  fb(2 * HID, HID), fb(1, HID), fb(HID, HID), fb(1, HID),
                  fb(HID, HID), fb(1, HID)],
        out_specs=[nb(HID), nb(XP)],
        out_shape=[jax.ShapeDtypeStruct((NN, HID), _f32),
                   jax.ShapeDtypeStruct((NN, XP), _f32)],
    )(h, xp, aggp, tp3, wn1, bn1, wn2, bn2, wo, bo)


def _row128(v):
    return v.reshape(1, HID)


def _pad8(w):
    return jnp.pad(w, ((0, 0), (0, 7)))


def kernel(h, x, edge_index, edge_attr, params):
    p = params
    ei = edge_index.astype(jnp.int32)
    row = ei[0]
    col = ei[1]
    xpad = jnp.zeros((NN, XP), _f32).at[:, :3].set(x)

    hcur = _pre_call(h, p["emb_in"]["w"], _row128(p["emb_in"]["b"]))

    lps = p["layers"]
    for i in range(len(lps)):
        lp = lps[i]
        xflat = jnp.pad(xpad[:, :4].reshape(-1), (0, 16))
        gr, gc, cdr = _sc_gather(hcur, row, col, xflat)
        m, t = _edge_call(
            gr, gc, cdr, edge_attr,
            lp["edge1"]["w"], _row128(lp["edge1"]["b"]),
            lp["edge2"]["w"], _row128(lp["edge2"]["b"]),
            _pad8(lp["att"]["w"]),
            jnp.full((1, HID), lp["att"]["b"][0], _f32),
            lp["coord1"]["w"], _row128(lp["coord1"]["b"]),
            _pad8(lp["coord2"]["w"]))
        aggp, tp = _sc_scatter(m, t, row)
        tp3 = tp[:, :4 * NN].reshape(NW, NN, 4)
        final = i == len(lps) - 1
        wo = p["emb_out"]["w"] if final else lp["node2"]["w"]
        bo = _row128(p["emb_out"]["b"]) if final else _row128(lp["node2"]["b"])
        hcur, xpad = _node_call(
            final, hcur, xpad, aggp, tp3,
            lp["node1"]["w"], _row128(lp["node1"]["b"]),
            lp["node2"]["w"], _row128(lp["node2"]["b"]), wo, bo)

    return (hcur, xpad[:, :3])


# speedup vs baseline: 1.0139x; 1.0139x over previous
import functools

import jax
import jax.numpy as jnp
from jax import lax
from jax.experimental import pallas as pl
from jax.experimental.pallas import tpu as pltpu
from jax.experimental.pallas import tpu_sc as plsc

NN = 10000
EE = 320000
HID = 128
DE = 16
XP = 16
XF = 4 * NN + 16

NC, NS = 2, 16
NW = NC * NS
EPW = EE // NW
CHUNK = 80
NCHUNK = EPW // CHUNK
S0 = 624
S_LAST = NN - S0 * (NS - 1)
ZROWS = 64

BE = 512
BN = 1000

_f32 = jnp.float32


@functools.lru_cache(maxsize=None)
def _sc_mesh():
    return plsc.VectorSubcoreMesh(
        core_axis_name="c", subcore_axis_name="s",
        num_cores=NC, num_subcores=NS)


def _gather_body(h_hbm, row_hbm, col_hbm, xf_hbm, gr_hbm, gc_hbm, cd_hbm,
                 idxr0, idxc0, bufR0, bufC0, cdb0,
                 idxr1, idxc1, bufR1, bufC1, cdb1, xvm,
                 semi0, semg0, semw0, semi1, semg1, semw1):
    cid = lax.axis_index("c")
    sid = lax.axis_index("s")
    wid = sid * NC + cid
    base = wid * EPW

    pltpu.sync_copy(xf_hbm, xvm)
    iota = lax.iota(jnp.int32, 16)
    lt3 = iota < 3

    sets = ((idxr0, idxc0, bufR0, bufC0, cdb0, semi0, semg0, semw0),
            (idxr1, idxc1, bufR1, bufC1, cdb1, semi1, semg1, semw1))

    def fire_idx(s, k):
        idxr, idxc, bufR, bufC, cdb, semi, semg, semw = sets[s]
        e0 = base + k * CHUNK
        i1 = pltpu.async_copy(row_hbm.at[pl.ds(e0, CHUNK)], idxr, semi)
        i2 = pltpu.async_copy(col_hbm.at[pl.ds(e0, CHUNK)], idxc, semi)
        return i1, i2

    def fire_gather(s):
        idxr, idxc, bufR, bufC, cdb, semi, semg, semw = sets[s]
        pltpu.async_copy(h_hbm.at[idxr], bufR, semg)
        pltpu.async_copy(h_hbm.at[idxc], bufC, semg)

    def fire_idx_and_gather(s, k):
        i1, i2 = fire_idx(s, k)
        i1.wait(); i2.wait()
        fire_gather(s)

    def wait_gather(s):
        idxr, idxc, bufR, bufC, cdb, semi, semg, semw = sets[s]
        pltpu.make_async_copy(h_hbm.at[idxr], bufR, semg).wait()
        pltpu.make_async_copy(h_hbm.at[idxc], bufC, semg).wait()

    def compute_and_write(s, k):
        idxr, idxc, bufR, bufC, cdb, semi, semg, semw = sets[s]
        e0 = base + k * CHUNK
        w1 = pltpu.async_copy(bufR, gr_hbm.at[pl.ds(e0, CHUNK)], semw)
        w2 = pltpu.async_copy(bufC, gc_hbm.at[pl.ds(e0, CHUNK)], semw)

        def egroup(g, c):
            rv = idxr[pl.ds(g * 16, 16)]
            cv = idxc[pl.ds(g * 16, 16)]
            for j in range(16):
                xr = plsc.load_gather(xvm, [rv[j] * 4 + iota])
                xc = plsc.load_gather(xvm, [cv[j] * 4 + iota])
                cdb[g * 16 + j, :] = jnp.where(lt3, xr - xc, 0.0)
            return c
        lax.fori_loop(0, CHUNK // 16, egroup, 0)

        w3 = pltpu.async_copy(cdb, cd_hbm.at[pl.ds(e0, CHUNK)], semw)
        return w1, w2, w3

    fire_idx_and_gather(0, 0)
    fire_idx_and_gather(1, 1)

    def pair(ko, carry):
        a = 2 * ko
        for s in range(2):
            k = a + s
            wait_gather(s)
            ws = compute_and_write(s, k)

            @pl.when(k + 2 < NCHUNK)
            def _():
                i1, i2 = fire_idx(s, k + 2)
                for w in ws:
                    w.wait()
                i1.wait(); i2.wait()
                fire_gather(s)

            @pl.when(k + 2 >= NCHUNK)
            def _():
                for w in ws:
                    w.wait()
        return carry

    lax.fori_loop(0, NCHUNK // 2, pair, 0)

    wait_gather(0)
    for w in compute_and_write(0, NCHUNK - 1):
        w.wait()


@functools.lru_cache(maxsize=None)
def _build_gather():
    bufset = [
        pltpu.VMEM((CHUNK,), jnp.int32),
        pltpu.VMEM((CHUNK,), jnp.int32),
        pltpu.VMEM((CHUNK, HID), _f32),
        pltpu.VMEM((CHUNK, HID), _f32),
        pltpu.VMEM((CHUNK, XP), _f32),
    ]
    return pl.kernel(
        _gather_body,
        out_type=(jax.ShapeDtypeStruct((EE, HID), _f32),
                  jax.ShapeDtypeStruct((EE, HID), _f32),
                  jax.ShapeDtypeStruct((EE, XP), _f32)),
        mesh=_sc_mesh(),
        scratch_types=bufset + bufset + [
            pltpu.VMEM((XF,), _f32),
            pltpu.SemaphoreType.DMA,
            pltpu.SemaphoreType.DMA,
            pltpu.SemaphoreType.DMA,
            pltpu.SemaphoreType.DMA,
            pltpu.SemaphoreType.DMA,
            pltpu.SemaphoreType.DMA,
        ],
        compiler_params=pltpu.CompilerParams(needs_layout_passes=False),
    )


def _sc_gather(h, row, col, xflat):
    return _build_gather()(h, row, col, xflat)


def _scatter_m_body(m_hbm, row_hbm, aggp_hbm, idxv, mbuf, idxv1, mbuf1,
                    zb, shA, sem, sem1, sems, sems1):
    cid = lax.axis_index("c")
    sid = lax.axis_index("s")
    wid = sid * NC + cid
    base = wid * EPW
    r0 = sid * S0

    z16 = jnp.zeros((16,), _f32)

    def zrow(i, c):
        for j in range(HID // 16):
            zb[i, pl.ds(j * 16, 16)] = z16
        return c
    lax.fori_loop(0, ZROWS, zrow, 0)
    nfull = jnp.where(sid == NS - 1, S_LAST // ZROWS, S0 // ZROWS)

    def zcopy(q, c):
        pltpu.sync_copy(zb, shA.at[pl.ds(r0 + q * ZROWS, ZROWS)])
        return c
    lax.fori_loop(0, nfull, zcopy, 0)

    @pl.when(sid < NS - 1)
    def _():
        tail = S0 - (S0 // ZROWS) * ZROWS
        t0 = r0 + (S0 // ZROWS) * ZROWS
        pltpu.sync_copy(zb.at[pl.ds(0, tail)], shA.at[pl.ds(t0, tail)])

    plsc.subcore_barrier()

    sets = ((idxv, mbuf, sem, sems), (idxv1, mbuf1, sem1, sems1))

    def fire_load(s, k):
        iv, mb, sm, ss = sets[s]
        e0 = base + k * CHUNK
        pltpu.async_copy(row_hbm.at[pl.ds(e0, CHUNK)], iv, sm)
        pltpu.async_copy(m_hbm.at[pl.ds(e0, CHUNK)], mb, sm)

    def wait_load(s):
        iv, mb, sm, ss = sets[s]
        pltpu.make_async_copy(row_hbm.at[pl.ds(0, CHUNK)], iv, sm).wait()
        pltpu.make_async_copy(m_hbm.at[pl.ds(0, CHUNK)], mb, sm).wait()

    def do_scatter(s):
        iv, mb, sm, ss = sets[s]
        pltpu.sync_copy(mb, shA.at[iv], add=True)

    fire_load(0, 0)
    fire_load(1, 1)

    def pair(ko, carry):
        a = 2 * ko
        for s in range(2):
            k = a + s
            wait_load(s)
            do_scatter(s)

            @pl.when(k + 2 < NCHUNK)
            def _():
                fire_load(s, k + 2)
        return carry
    lax.fori_loop(0, NCHUNK // 2, pair, 0)

    wait_load(0)
    do_scatter(0)

    plsc.subcore_barrier()

    @pl.when(sid < NS - 1)
    def _():
        sl = pl.ds(r0, S0)
        pltpu.sync_copy(shA.at[sl], aggp_hbm.at[cid, sl])

    @pl.when(sid == NS - 1)
    def _():
        sl = pl.ds(r0, S_LAST)
        pltpu.sync_copy(shA.at[sl], aggp_hbm.at[cid, sl])


@functools.lru_cache(maxsize=None)
def _build_scatter_m():
    return pl.kernel(
        _scatter_m_body,
        out_type=jax.ShapeDtypeStruct((NC, NN, HID), _f32),
        mesh=_sc_mesh(),
        scratch_types=[
            pltpu.VMEM((CHUNK,), jnp.int32),
            pltpu.VMEM((CHUNK, HID), _f32),
            pltpu.VMEM((CHUNK,), jnp.int32),
            pltpu.VMEM((CHUNK, HID), _f32),
            pltpu.VMEM((ZROWS, HID), _f32),
            pltpu.VMEM_SHARED((NN, HID), _f32),
            pltpu.SemaphoreType.DMA,
            pltpu.SemaphoreType.DMA,
            pltpu.SemaphoreType.DMA,
            pltpu.SemaphoreType.DMA,
        ],
    )


def _scatter_t_body(t_hbm, row_hbm, tp_hbm, idxv, tbuf, idxv1, tbuf1,
                    acc, sem, sem1):
    cid = lax.axis_index("c")
    sid = lax.axis_index("s")
    wid = sid * NC + cid
    base = wid * EPW

    z16 = jnp.zeros((16,), _f32)

    def zrow(i, c):
        acc[pl.ds(i * 16, 16)] = z16
        return c
    lax.fori_loop(0, XF // 16, zrow, 0)

    iota = lax.iota(jnp.int32, 16)
    lt4 = iota < 4

    sets = ((idxv, tbuf, sem), (idxv1, tbuf1, sem1))

    def fire_load(s, k):
        iv, tb, sm = sets[s]
        e0 = base + k * CHUNK
        pltpu.async_copy(row_hbm.at[pl.ds(e0, CHUNK)], iv, sm)
        pltpu.async_copy(t_hbm.at[pl.ds(e0, CHUNK)], tb, sm)

    def wait_load(s):
        iv, tb, sm = sets[s]
        pltpu.make_async_copy(row_hbm.at[pl.ds(0, CHUNK)], iv, sm).wait()
        pltpu.make_async_copy(t_hbm.at[pl.ds(0, CHUNK)], tb, sm).wait()

    def do_scatter(s):
        iv, tb, sm = sets[s]

        def egroup(g, c):
            rv = iv[pl.ds(g * 16, 16)]
            for j in range(16):
                plsc.addupdate_scatter(
                    acc, [rv[j] * 4 + iota], tb[g * 16 + j, :], mask=lt4)
            return c
        lax.fori_loop(0, CHUNK // 16, egroup, 0)

    fire_load(0, 0)
    fire_load(1, 1)

    def pair(ko, carry):
        a = 2 * ko
        for s in range(2):
            k = a + s
            wait_load(s)
            do_scatter(s)

            @pl.when(k + 2 < NCHUNK)
            def _():
                fire_load(s, k + 2)
        return carry
    lax.fori_loop(0, NCHUNK // 2, pair, 0)

    wait_load(0)
    do_scatter(0)

    pltpu.sync_copy(acc, tp_hbm.at[wid])


@functools.lru_cache(maxsize=None)
def _build_scatter_t():
    return pl.kernel(
        _scatter_t_body,
        out_type=jax.ShapeDtypeStruct((NW, XF), _f32),
        mesh=_sc_mesh(),
        scratch_types=[
            pltpu.VMEM((CHUNK,), jnp.int32),
            pltpu.VMEM((CHUNK, XP), _f32),
            pltpu.VMEM((CHUNK,), jnp.int32),
            pltpu.VMEM((CHUNK, XP), _f32),
            pltpu.VMEM((XF,), _f32),
            pltpu.SemaphoreType.DMA,
            pltpu.SemaphoreType.DMA,
        ],
        compiler_params=pltpu.CompilerParams(needs_layout_passes=False),
    )


def _sc_scatter(m, t, row):
    aggp = _build_scatter_m()(m, row)
    tp = _build_scatter_t()(t, row)
    return aggp, tp


def _silu(v):
    return v * jax.nn.sigmoid(v)


def _edge_body(gr_ref, gc_ref, cd_ref, ea_ref, w1, b1, w2, b2, wa8, ba,
               wc1, bc1, wc28, m_out, t_out):
    gr = gr_ref[...]
    gc = gc_ref[...]
    cd = cd_ref[...]
    ea = ea_ref[...]
    radial = jnp.sum(cd * cd, axis=1, keepdims=True)
    m1 = jnp.concatenate([gr, gc, radial, ea], axis=1)
    m = _silu(jnp.dot(m1, w1[...], preferred_element_type=_f32) + b1[...])
    m = _silu(jnp.dot(m, w2[...], preferred_element_type=_f32) + b2[...])
    att = jax.nn.sigmoid(
        jnp.dot(m, wa8[...], preferred_element_type=_f32)[:, :1]
        + ba[...][:, :1])
    mo = m * att
    phi = _silu(jnp.dot(mo, wc1[...], preferred_element_type=_f32) + bc1[...])
    p2 = jnp.tanh(
        jnp.dot(phi, wc28[...], preferred_element_type=_f32)[:, :1])
    t = cd * p2
    lane = lax.broadcasted_iota(jnp.int32, t.shape, 1)
    t = jnp.where(lane == 3, 1.0, t)
    m_out[...] = mo
    t_out[...] = t


def _edge_call(gr, gc, cd, ea, w1, b1, w2, b2, wa8, ba, wc1, bc1, wc28):
    eb = lambda w: pl.BlockSpec((BE, w), lambda i: (i, 0))
    fb = lambda a, b: pl.BlockSpec((a, b), lambda i: (0, 0))
    return pl.pallas_call(
        _edge_body,
        grid=(EE // BE,),
        in_specs=[eb(HID), eb(HID), eb(XP), eb(DE),
                  fb(2 * HID + 1 + DE, HID), fb(1, HID), fb(HID, HID),
                  fb(1, HID), fb(HID, 8), fb(1, HID), fb(HID, HID),
                  fb(1, HID), fb(HID, 8)],
        out_specs=[eb(HID), eb(XP)],
        out_shape=[jax.ShapeDtypeStruct((EE, HID), _f32),
                   jax.ShapeDtypeStruct((EE, XP), _f32)],
    )(gr, gc, cd, ea, w1, b1, w2, b2, wa8, ba, wc1, bc1, wc28)


def _pre_body(h_ref, win, bin_, h_out):
    h_out[...] = jnp.dot(h_ref[...], win[...],
                         preferred_element_type=_f32) + bin_[...]


def _pre_call(h, win, bin_):
    nb = lambda: pl.BlockSpec((BN, HID), lambda i: (i, 0))
    fb = lambda a: pl.BlockSpec((a, HID), lambda i: (0, 0))
    return pl.pallas_call(
        _pre_body,
        grid=(NN // BN,),
        in_specs=[nb(), fb(HID), fb(1)],
        out_specs=nb(),
        out_shape=jax.ShapeDtypeStruct((NN, HID), _f32),
    )(h, win, bin_)


def _node_body(final, h_ref, xp_ref, aggp_ref, tp_ref,
               wn1, bn1, wn2, bn2, wo, bo, *outs):
    h = h_ref[...]
    xp = xp_ref[...]
    aggp = aggp_ref[...]
    agg = aggp[0] + aggp[1]
    s4 = jnp.sum(tp_ref[...], axis=0)
    cnt = jnp.clip(s4[:, 3:4], 1.0, None)
    upd3 = s4[:, :3] / cnt
    xn = xp + jnp.concatenate(
        [upd3, jnp.zeros((upd3.shape[0], XP - 3), _f32)], axis=1)
    z1 = jnp.concatenate([h, agg], axis=1)
    z = _silu(jnp.dot(z1, wn1[...], preferred_element_type=_f32) + bn1[...])
    z = jnp.dot(z, wn2[...], preferred_element_type=_f32) + bn2[...]
    hn = h + z
    h_out, x_out = outs
    if final:
        hn = jnp.dot(hn, wo[...], preferred_element_type=_f32) + bo[...]
    h_out[...] = hn
    x_out[...] = xn


def _node_call(final, h, xp, aggp, tp3, wn1, bn1, wn2, bn2, wo, bo):
    nb = lambda w: pl.BlockSpec((BN, w), lambda i: (i, 0))
    fb = lambda a, b: pl.BlockSpec((a, b), lambda i: (0, 0))
    return pl.pallas_call(
        functools.partial(_node_body, final),
        grid=(NN // BN,),
        in_specs=[nb(HID), nb(XP),
                  pl.BlockSpec((NC, BN, HID), lambda i: (0, i, 0)),
                  pl.BlockSpec((NW, BN, 4), lambda i: (0, i, 0)),
                  fb(2 * HID, HID), fb(1, HID), fb(HID, HID), fb(1, HID),
                  fb(HID, HID), fb(1, HID)],
        out_specs=[nb(HID), nb(XP)],
        out_shape=[jax.ShapeDtypeStruct((NN, HID), _f32),
                   jax.ShapeDtypeStruct((NN, XP), _f32)],
    )(h, xp, aggp, tp3, wn1, bn1, wn2, bn2, wo, bo)


def _row128(v):
    return v.reshape(1, HID)


def _pad8(w):
    return jnp.pad(w, ((0, 0), (0, 7)))


def kernel(h, x, edge_index, edge_attr, params):
    p = params
    ei = edge_index.astype(jnp.int32)
    row = ei[0]
    col = ei[1]
    xpad = jnp.zeros((NN, XP), _f32).at[:, :3].set(x)

    hcur = _pre_call(h, p["emb_in"]["w"], _row128(p["emb_in"]["b"]))

    lps = p["layers"]
    for i in range(len(lps)):
        lp = lps[i]
        xflat = jnp.pad(xpad[:, :4].reshape(-1), (0, 16))
        gr, gc, cdr = _sc_gather(hcur, row, col, xflat)
        m, t = _edge_call(
            gr, gc, cdr, edge_attr,
            lp["edge1"]["w"], _row128(lp["edge1"]["b"]),
            lp["edge2"]["w"], _row128(lp["edge2"]["b"]),
            _pad8(lp["att"]["w"]),
            jnp.full((1, HID), lp["att"]["b"][0], _f32),
            lp["coord1"]["w"], _row128(lp["coord1"]["b"]),
            _pad8(lp["coord2"]["w"]))
        aggp, tp = _sc_scatter(m, t, row)
        tp3 = tp[:, :4 * NN].reshape(NW, NN, 4)
        final = i == len(lps) - 1
        wo = p["emb_out"]["w"] if final else lp["node2"]["w"]
        bo = _row128(p["emb_out"]["b"]) if final else _row128(lp["node2"]["b"])
        hcur, xpad = _node_call(
            final, hcur, xpad, aggp, tp3,
            lp["node1"]["w"], _row128(lp["node1"]["b"]),
            lp["node2"]["w"], _row128(lp["node2"]["b"]), wo, bo)

    return (hcur, xpad[:, :3])
